# baseline (speedup 1.0000x reference)
import numpy as np
import jax
import jax.numpy as jnp
from jax import lax
from jax.experimental import pallas as pl
from jax.experimental.pallas import tpu as pltpu
from jax.experimental.pallas import tpu_sc as plsc

B, N, C_IN, C_HID, C_OUT = 8, 1568, 192, 768, 192
H, W = 28, 28
NPIX = H * W
CH2 = C_HID // 2
CPW = CH2 + 32
C_PAD = 256

W4 = float(np.float32(1.0) / (np.float32(4.0) + np.float32(1e-6)))

_GW = 28
_GPW = (B * NPIX) // _GW
_GCH = 32
_GNCH = _GPW // _GCH
_SPT = 49
_RS = 98


def _pos_table():
    p = np.arange(NPIX)
    pr, pc = p // W, p % W
    out = np.empty((NPIX, 4), np.int64)
    k = 0
    for dr in (0, 1):
        for dc in (0, 1):
            out[:, k] = (2 * pr + dr) * (2 * W) + (2 * pc + dc)
            k += 1
    return out


_POS = _pos_table()


def _sc_gather_body(x_hbm, gidx_hbm, out_hbm, idx_v, rows_a, rows_b, acc_v,
                    sem_a, sem_b):
    cid = lax.axis_index("c")
    sid = lax.axis_index("s")
    wid = sid * 2 + cid

    @pl.when(wid < _GW)
    def _():
        pltpu.sync_copy(gidx_hbm.at[wid], idx_v)
        bufs = (rows_a, rows_b)
        sems = (sem_a, sem_b)
        handles = [None] * _GNCH
        handles[0] = pltpu.async_copy(x_hbm.at[idx_v.at[0]], rows_a, sem_a)
        for ck in range(_GNCH):
            handles[ck].wait()
            if ck + 1 < _GNCH:
                handles[ck + 1] = pltpu.async_copy(
                    x_hbm.at[idx_v.at[ck + 1]], bufs[(ck + 1) % 2],
                    sems[(ck + 1) % 2])
            cur = bufs[ck % 2]

            def body(p, carry, cur=cur):
                for cc in range(C_IN // 16):
                    sl = pl.ds(cc * 16, 16)
                    v = (cur[4 * p, sl] + cur[4 * p + 1, sl]) + (
                        cur[4 * p + 2, sl] + cur[4 * p + 3, sl])
                    acc_v[p, sl] = v * W4
                return carry

            lax.fori_loop(0, _GCH, body, 0)
            pltpu.sync_copy(acc_v,
                            out_hbm.at[pl.ds(wid * _GPW + ck * _GCH, _GCH)])


def _sc_scatter_body(mapc_hbm, tok4_hbm, out_hbm, buf_a, buf_b, tidx_v, zero_v,
                     acc_sh, sem_a, sem_b, sem_s):
    cid = lax.axis_index("c")
    sid = lax.axis_index("s")

    def zbody(r, carry):
        for cc in range(CPW // 32):
            zero_v[r, pl.ds(cc * 32, 32)] = jnp.zeros((32,), jnp.bfloat16)
        return carry

    lax.fori_loop(0, _SPT, zbody, 0)
    pltpu.sync_copy(zero_v, acc_sh.at[pl.ds(sid * _RS, _SPT)])
    pltpu.sync_copy(zero_v, acc_sh.at[pl.ds(sid * _RS + _SPT, _SPT)])

    bufs = (buf_a, buf_b)
    sems = (sem_a, sem_b)
    nseg = B // 2 * 2
    loads = [None] * nseg
    b0 = cid * (B // 2)
    loads[0] = pltpu.async_copy(
        mapc_hbm.at[b0, 0, pl.ds(sid * _SPT, _SPT)], buf_a, sem_a)
    for seg in range(nseg):
        bb, p = seg // 2, seg % 2
        b = b0 + bb
        if p == 0:
            pltpu.sync_copy(tok4_hbm.at[b, sid], tidx_v)
        loads[seg].wait()
        plsc.subcore_barrier()
        cur = bufs[seg % 2]
        scat = [pltpu.async_copy(cur, acc_sh.at[tidx_v.at[j]], sem_s, add=True)
                for j in range(4)]
        if seg + 1 < nseg:
            nb, np_ = (seg + 1) // 2, (seg + 1) % 2
            loads[seg + 1] = pltpu.async_copy(
                mapc_hbm.at[b0 + nb, np_, pl.ds(sid * _SPT, _SPT)],
                bufs[(seg + 1) % 2], sems[(seg + 1) % 2])
        for h in scat:
            h.wait()
        plsc.subcore_barrier()
        pltpu.sync_copy(acc_sh.at[pl.ds(sid * _RS, _RS)],
                        out_hbm.at[b, p, pl.ds(sid * _RS, _RS)])
        pltpu.sync_copy(zero_v, acc_sh.at[pl.ds(sid * _RS, _SPT)])
        pltpu.sync_copy(zero_v, acc_sh.at[pl.ds(sid * _RS + _SPT, _SPT)])


def _tc_map_body(xmap_ref, fc1t_ref, fc1b_ref, dwt_ref, dwb_ref, out_ref):
    m = jnp.dot(xmap_ref[0].astype(jnp.bfloat16),
                fc1t_ref[...].astype(jnp.bfloat16),
                preferred_element_type=jnp.float32) + fc1b_ref[...]
    m3 = m.reshape(H, W, C_HID)
    zr = jnp.zeros((1, W, C_HID), jnp.float32)
    t = jnp.concatenate([zr, m3, zr], axis=0)
    zc = jnp.zeros((H + 2, 1, C_HID), jnp.float32)
    t = jnp.concatenate([zc, t, zc], axis=1)
    acc = jnp.zeros((H, W, C_HID), jnp.float32)
    for kh in range(3):
        for kw in range(3):
            acc = acc + t[kh:kh + H, kw:kw + W, :] * dwt_ref[3 * kh + kw][None, None, :]
    acc = acc + dwb_ref[...][None]
    res = acc.reshape(NPIX, C_HID)
    pat = jnp.concatenate(
        [jnp.ones((NPIX, 1), jnp.float32), jnp.zeros((NPIX, 31), jnp.float32)],
        axis=1)
    out_ref[0, 0] = jnp.concatenate([res[:, :CH2], pat], axis=1).astype(jnp.bfloat16)
    out_ref[0, 1] = jnp.concatenate(
        [res[:, CH2:], jnp.zeros((NPIX, 32), jnp.float32)], axis=1).astype(jnp.bfloat16)


def _tc_skip_body(x_ref, fc1t_ref, fc1b_ref, skip_ref, out_ref):
    h = jnp.dot(x_ref[0].astype(jnp.bfloat16), fc1t_ref[...],
                preferred_element_type=jnp.float32) + fc1b_ref[...]
    out_ref[0] = (h * skip_ref[...]).astype(jnp.bfloat16)


def _tc_out_body(hs_ref, tok_ref, fc2t_ref, fc2b_ref, out_ref):
    cnt = lax.slice(tok_ref[0, 0], (0, CH2), (N, CH2 + 1)).astype(jnp.float32)
    val = 1.0 / (cnt + 1e-6)
    tokv = jnp.concatenate(
        [lax.slice(tok_ref[0, 0], (0, 0), (N, CH2)),
         lax.slice(tok_ref[0, 1], (0, 0), (N, CH2))], axis=1).astype(jnp.float32)
    g = hs_ref[0].astype(jnp.float32) + tokv * val
    gel = 0.5 * g * (1.0 + lax.erf(g * np.float32(1.0 / np.sqrt(2.0))))
    out_ref[0] = jnp.dot(gel.astype(jnp.bfloat16), fc2t_ref[...],
                         preferred_element_type=jnp.float32) + fc2b_ref[...]


def kernel(x, idx_token, fc1_w, fc1_b, skip_w, dw_w, dw_b, fc2_w, fc2_b):
    pos = jnp.asarray(_POS.reshape(-1), dtype=jnp.int32)
    a2 = jnp.take(idx_token, pos, axis=1).reshape(B, NPIX, 4).astype(jnp.int32)
    goff = a2 + (jnp.arange(B, dtype=jnp.int32) * N)[:, None, None]
    gidx = goff.reshape(_GW, _GNCH, 4 * _GCH)
    tok4 = a2.reshape(B, 16, _SPT, 4).transpose(0, 1, 3, 2)

    mesh = plsc.VectorSubcoreMesh(core_axis_name="c", subcore_axis_name="s")

    xmap = pl.kernel(
        _sc_gather_body,
        out_type=jax.ShapeDtypeStruct((B * NPIX, C_IN), jnp.float32),
        mesh=mesh,
        scratch_types=[
            pltpu.VMEM((_GNCH, 4 * _GCH), jnp.int32),
            pltpu.VMEM((4 * _GCH, C_PAD), jnp.float32),
            pltpu.VMEM((4 * _GCH, C_PAD), jnp.float32),
            pltpu.VMEM((_GCH, C_IN), jnp.float32),
            pltpu.SemaphoreType.DMA,
            pltpu.SemaphoreType.DMA,
        ],
    )(jnp.pad(x.reshape(B * N, C_IN), ((0, 0), (0, C_PAD - C_IN))), gidx)

    mapc = pl.pallas_call(
        _tc_map_body,
        grid=(B,),
        in_specs=[
            pl.BlockSpec((1, NPIX, C_IN), lambda b: (b, 0, 0)),
            pl.BlockSpec((C_IN, C_HID), lambda b: (0, 0)),
            pl.BlockSpec((1, C_HID), lambda b: (0, 0)),
            pl.BlockSpec((9, C_HID), lambda b: (0, 0)),
            pl.BlockSpec((1, C_HID), lambda b: (0, 0)),
        ],
        out_specs=pl.BlockSpec((1, 2, NPIX, CPW), lambda b: (b, 0, 0, 0)),
        out_shape=jax.ShapeDtypeStruct((B, 2, NPIX, CPW), jnp.bfloat16),
    )(
        xmap.reshape(B, NPIX, C_IN),
        fc1_w.T,
        fc1_b.reshape(1, C_HID),
        dw_w.reshape(C_HID, 9).T,
        dw_b.reshape(1, C_HID),
    )

    tok_acc = pl.kernel(
        _sc_scatter_body,
        out_type=jax.ShapeDtypeStruct((B, 2, N, CPW), jnp.bfloat16),
        mesh=mesh,
        compiler_params=pltpu.CompilerParams(use_tc_tiling_on_sc=False),
        scratch_types=[
            pltpu.VMEM((_SPT, CPW), jnp.bfloat16),
            pltpu.VMEM((_SPT, CPW), jnp.bfloat16),
            pltpu.VMEM((4, _SPT), jnp.int32),
            pltpu.VMEM((_SPT, CPW), jnp.bfloat16),
            pltpu.VMEM_SHARED((N, CPW), jnp.bfloat16),
            pltpu.SemaphoreType.DMA,
            pltpu.SemaphoreType.DMA,
            pltpu.SemaphoreType.DMA,
        ],
    )(mapc, tok4)

    h_skip = pl.pallas_call(
        _tc_skip_body,
        grid=(B,),
        in_specs=[
            pl.BlockSpec((1, N, C_IN), lambda b: (b, 0, 0)),
            pl.BlockSpec((C_IN, C_HID), lambda b: (0, 0)),
            pl.BlockSpec((1, C_HID), lambda b: (0, 0)),
            pl.BlockSpec((1, C_HID), lambda b: (0, 0)),
        ],
        out_specs=pl.BlockSpec((1, N, C_HID), lambda b: (b, 0, 0)),
        out_shape=jax.ShapeDtypeStruct((B, N, C_HID), jnp.bfloat16),
    )(
        x,
        fc1_w.T.astype(jnp.bfloat16),
        fc1_b.reshape(1, C_HID),
        skip_w.reshape(1, C_HID),
    )

    out = pl.pallas_call(
        _tc_out_body,
        grid=(B,),
        in_specs=[
            pl.BlockSpec((1, N, C_HID), lambda b: (b, 0, 0)),
            pl.BlockSpec((1, 2, N, CPW), lambda b: (b, 0, 0, 0)),
            pl.BlockSpec((C_HID, C_OUT), lambda b: (0, 0)),
            pl.BlockSpec((1, C_OUT), lambda b: (0, 0)),
        ],
        out_specs=pl.BlockSpec((1, N, C_OUT), lambda b: (b, 0, 0)),
        out_shape=jax.ShapeDtypeStruct((B, N, C_OUT), jnp.float32),
    )(
        h_skip,
        tok_acc,
        fc2_w.T.astype(jnp.bfloat16),
        fc2_b.reshape(1, C_OUT),
    )
    return out


# speedup vs baseline: 1.0002x; 1.0002x over previous
import numpy as np
import jax
import jax.numpy as jnp
from jax import lax
from jax.experimental import pallas as pl
from jax.experimental.pallas import tpu as pltpu
from jax.experimental.pallas import tpu_sc as plsc

B, N, C_IN, C_HID, C_OUT = 8, 1568, 192, 768, 192
H, W = 28, 28
NPIX = H * W
CH2 = C_HID // 2
CPW = CH2 + 16
C_PAD = 256

W4 = float(np.float32(1.0) / (np.float32(4.0) + np.float32(1e-6)))

_GW = 28
_GPW = (B * NPIX) // _GW
_GCH = 32
_GNCH = _GPW // _GCH
_SPT = 49
_RS = 98


def _pos_table():
    p = np.arange(NPIX)
    pr, pc = p // W, p % W
    out = np.empty((NPIX, 4), np.int64)
    k = 0
    for dr in (0, 1):
        for dc in (0, 1):
            out[:, k] = (2 * pr + dr) * (2 * W) + (2 * pc + dc)
            k += 1
    return out


_POS = _pos_table()


def _sc_gather_body(x_hbm, gidx_hbm, out_hbm, idx_v, rows_a, rows_b, acc_v,
                    sem_a, sem_b):
    cid = lax.axis_index("c")
    sid = lax.axis_index("s")
    wid = sid * 2 + cid

    @pl.when(wid < _GW)
    def _():
        pltpu.sync_copy(gidx_hbm.at[wid], idx_v)
        bufs = (rows_a, rows_b)
        sems = (sem_a, sem_b)
        handles = [None] * _GNCH
        handles[0] = pltpu.async_copy(x_hbm.at[idx_v.at[0]], rows_a, sem_a)
        for ck in range(_GNCH):
            handles[ck].wait()
            if ck + 1 < _GNCH:
                handles[ck + 1] = pltpu.async_copy(
                    x_hbm.at[idx_v.at[ck + 1]], bufs[(ck + 1) % 2],
                    sems[(ck + 1) % 2])
            cur = bufs[ck % 2]

            def body(p, carry, cur=cur):
                for cc in range(C_IN // 16):
                    sl = pl.ds(cc * 16, 16)
                    v = (cur[4 * p, sl] + cur[4 * p + 1, sl]) + (
                        cur[4 * p + 2, sl] + cur[4 * p + 3, sl])
                    acc_v[p, sl] = v * W4
                return carry

            lax.fori_loop(0, _GCH, body, 0)
            pltpu.sync_copy(acc_v,
                            out_hbm.at[pl.ds(wid * _GPW + ck * _GCH, _GCH)])


def _sc_scatter_body(mapc_hbm, tok4_hbm, out_hbm, buf_a, buf_b, tidx_v, zero_v,
                     acc_sh, sem_a, sem_b, sem_s):
    cid = lax.axis_index("c")
    sid = lax.axis_index("s")

    def zbody(r, carry):
        for cc in range(CPW // 16):
            zero_v[r, pl.ds(cc * 16, 16)] = jnp.zeros((16,), jnp.float32)
        return carry

    lax.fori_loop(0, _SPT, zbody, 0)
    pltpu.sync_copy(zero_v, acc_sh.at[pl.ds(sid * _RS, _SPT)])
    pltpu.sync_copy(zero_v, acc_sh.at[pl.ds(sid * _RS + _SPT, _SPT)])

    bufs = (buf_a, buf_b)
    sems = (sem_a, sem_b)
    nseg = 4
    loads = [None] * nseg
    b0 = cid * 2
    loads[0] = pltpu.async_copy(
        mapc_hbm.at[b0, 0, pl.ds(sid * _SPT, _SPT)], buf_a, sem_a)
    for seg in range(nseg):
        bb, p = seg // 2, seg % 2
        b = b0 + bb
        if p == 0:
            pltpu.sync_copy(tok4_hbm.at[b, sid], tidx_v)
        loads[seg].wait()
        plsc.subcore_barrier()
        cur = bufs[seg % 2]
        scat = [pltpu.async_copy(cur, acc_sh.at[tidx_v.at[j]], sem_s, add=True)
                for j in range(4)]
        if seg + 1 < nseg:
            nb, np_ = (seg + 1) // 2, (seg + 1) % 2
            loads[seg + 1] = pltpu.async_copy(
                mapc_hbm.at[b0 + nb, np_, pl.ds(sid * _SPT, _SPT)],
                bufs[(seg + 1) % 2], sems[(seg + 1) % 2])
        for h in scat:
            h.wait()
        plsc.subcore_barrier()
        pltpu.sync_copy(acc_sh.at[pl.ds(sid * _RS, _RS)],
                        out_hbm.at[b, p, pl.ds(sid * _RS, _RS)])
        pltpu.sync_copy(zero_v, acc_sh.at[pl.ds(sid * _RS, _SPT)])
        pltpu.sync_copy(zero_v, acc_sh.at[pl.ds(sid * _RS + _SPT, _SPT)])


def _tc_map_body(xmap_ref, fc1t_ref, fc1b_ref, dwt_ref, dwb_ref, out_ref):
    m = jnp.dot(xmap_ref[0].astype(jnp.bfloat16),
                fc1t_ref[...].astype(jnp.bfloat16),
                preferred_element_type=jnp.float32) + fc1b_ref[...]
    m3 = m.reshape(H, W, C_HID)
    zr = jnp.zeros((1, W, C_HID), jnp.float32)
    t = jnp.concatenate([zr, m3, zr], axis=0)
    zc = jnp.zeros((H + 2, 1, C_HID), jnp.float32)
    t = jnp.concatenate([zc, t, zc], axis=1)
    acc = jnp.zeros((H, W, C_HID), jnp.float32)
    for kh in range(3):
        for kw in range(3):
            acc = acc + t[kh:kh + H, kw:kw + W, :] * dwt_ref[3 * kh + kw][None, None, :]
    acc = acc + dwb_ref[...][None]
    res = acc.reshape(NPIX, C_HID)
    pat = jnp.concatenate(
        [jnp.ones((NPIX, 1), jnp.float32), jnp.zeros((NPIX, 15), jnp.float32)],
        axis=1)
    out_ref[0, 0] = jnp.concatenate([res[:, :CH2], pat], axis=1)
    out_ref[0, 1] = jnp.concatenate(
        [res[:, CH2:], jnp.zeros((NPIX, 16), jnp.float32)], axis=1)


def _tc_skip_body(x_ref, fc1t_ref, fc1b_ref, skip_ref, out_ref):
    h = jnp.dot(x_ref[0].astype(jnp.bfloat16), fc1t_ref[...],
                preferred_element_type=jnp.float32) + fc1b_ref[...]
    out_ref[0] = (h * skip_ref[...]).astype(jnp.bfloat16)


def _tc_out_body(hs_ref, tok_ref, fc2t_ref, fc2b_ref, out_ref):
    cnt = lax.slice(tok_ref[0, 0], (0, CH2), (N, CH2 + 1)).astype(jnp.float32)
    val = 1.0 / (cnt + 1e-6)
    tokv = jnp.concatenate(
        [lax.slice(tok_ref[0, 0], (0, 0), (N, CH2)),
         lax.slice(tok_ref[0, 1], (0, 0), (N, CH2))], axis=1).astype(jnp.float32)
    g = hs_ref[0].astype(jnp.float32) + tokv * val
    gel = 0.5 * g * (1.0 + lax.erf(g * np.float32(1.0 / np.sqrt(2.0))))
    out_ref[0] = jnp.dot(gel.astype(jnp.bfloat16), fc2t_ref[...],
                         preferred_element_type=jnp.float32) + fc2b_ref[...]


def kernel(x, idx_token, fc1_w, fc1_b, skip_w, dw_w, dw_b, fc2_w, fc2_b):
    pos = jnp.asarray(_POS.reshape(-1), dtype=jnp.int32)
    a2 = jnp.take(idx_token, pos, axis=1).reshape(B, NPIX, 4).astype(jnp.int32)
    goff = a2 + (jnp.arange(B, dtype=jnp.int32) * N)[:, None, None]
    gidx = goff.reshape(_GW, _GNCH, 4 * _GCH)
    tok4 = a2.reshape(B, 16, _SPT, 4).transpose(0, 1, 3, 2)

    mesh = plsc.VectorSubcoreMesh(core_axis_name="c", subcore_axis_name="s")

    xmap = pl.kernel(
        _sc_gather_body,
        out_type=jax.ShapeDtypeStruct((B * NPIX, C_IN), jnp.float32),
        mesh=mesh,
        scratch_types=[
            pltpu.VMEM((_GNCH, 4 * _GCH), jnp.int32),
            pltpu.VMEM((4 * _GCH, C_PAD), jnp.float32),
            pltpu.VMEM((4 * _GCH, C_PAD), jnp.float32),
            pltpu.VMEM((_GCH, C_IN), jnp.float32),
            pltpu.SemaphoreType.DMA,
            pltpu.SemaphoreType.DMA,
        ],
    )(jnp.pad(x.reshape(B * N, C_IN), ((0, 0), (0, C_PAD - C_IN))), gidx)

    xmap8 = xmap.reshape(B, NPIX, C_IN)
    hb = B // 2
    tok_halves = []
    for hh in range(2):
        mapc_h = pl.pallas_call(
            _tc_map_body,
            grid=(hb,),
            in_specs=[
                pl.BlockSpec((1, NPIX, C_IN), lambda b: (b, 0, 0)),
                pl.BlockSpec((C_IN, C_HID), lambda b: (0, 0)),
                pl.BlockSpec((1, C_HID), lambda b: (0, 0)),
                pl.BlockSpec((9, C_HID), lambda b: (0, 0)),
                pl.BlockSpec((1, C_HID), lambda b: (0, 0)),
            ],
            out_specs=pl.BlockSpec((1, 2, NPIX, CPW), lambda b: (b, 0, 0, 0)),
            out_shape=jax.ShapeDtypeStruct((hb, 2, NPIX, CPW), jnp.float32),
        )(
            lax.slice_in_dim(xmap8, hh * hb, (hh + 1) * hb, axis=0),
            fc1_w.T,
            fc1_b.reshape(1, C_HID),
            dw_w.reshape(C_HID, 9).T,
            dw_b.reshape(1, C_HID),
        )

        tok_h = pl.kernel(
            _sc_scatter_body,
            out_type=jax.ShapeDtypeStruct((hb, 2, N, CPW), jnp.float32),
            mesh=mesh,
            compiler_params=pltpu.CompilerParams(use_tc_tiling_on_sc=False),
            scratch_types=[
                pltpu.VMEM((_SPT, CPW), jnp.float32),
                pltpu.VMEM((_SPT, CPW), jnp.float32),
                pltpu.VMEM((4, _SPT), jnp.int32),
                pltpu.VMEM((_SPT, CPW), jnp.float32),
                pltpu.VMEM_SHARED((N, CPW), jnp.float32),
                pltpu.SemaphoreType.DMA,
                pltpu.SemaphoreType.DMA,
                pltpu.SemaphoreType.DMA,
            ],
        )(mapc_h, lax.slice_in_dim(tok4, hh * hb, (hh + 1) * hb, axis=0))
        tok_halves.append(tok_h)

    h_skip = pl.pallas_call(
        _tc_skip_body,
        grid=(B,),
        in_specs=[
            pl.BlockSpec((1, N, C_IN), lambda b: (b, 0, 0)),
            pl.BlockSpec((C_IN, C_HID), lambda b: (0, 0)),
            pl.BlockSpec((1, C_HID), lambda b: (0, 0)),
            pl.BlockSpec((1, C_HID), lambda b: (0, 0)),
        ],
        out_specs=pl.BlockSpec((1, N, C_HID), lambda b: (b, 0, 0)),
        out_shape=jax.ShapeDtypeStruct((B, N, C_HID), jnp.bfloat16),
    )(
        x,
        fc1_w.T.astype(jnp.bfloat16),
        fc1_b.reshape(1, C_HID),
        skip_w.reshape(1, C_HID),
    )

    outs = []
    for hh in range(2):
        out_h = pl.pallas_call(
            _tc_out_body,
            grid=(hb,),
            in_specs=[
                pl.BlockSpec((1, N, C_HID), lambda b: (b, 0, 0)),
                pl.BlockSpec((1, 2, N, CPW), lambda b: (b, 0, 0, 0)),
                pl.BlockSpec((C_HID, C_OUT), lambda b: (0, 0)),
                pl.BlockSpec((1, C_OUT), lambda b: (0, 0)),
            ],
            out_specs=pl.BlockSpec((1, N, C_OUT), lambda b: (b, 0, 0)),
            out_shape=jax.ShapeDtypeStruct((hb, N, C_OUT), jnp.float32),
        )(
            lax.slice_in_dim(h_skip, hh * hb, (hh + 1) * hb, axis=0),
            tok_halves[hh],
            fc2_w.T.astype(jnp.bfloat16),
            fc2_b.reshape(1, C_OUT),
        )
        outs.append(out_h)
    return jnp.concatenate(outs, axis=0)


# speedup vs baseline: 1.0917x; 1.0915x over previous
import numpy as np
import jax
import jax.numpy as jnp
from jax import lax
from jax.experimental import pallas as pl
from jax.experimental.pallas import tpu as pltpu
from jax.experimental.pallas import tpu_sc as plsc

B, N, C_IN, C_HID, C_OUT = 8, 1568, 192, 768, 192
H, W = 28, 28
NPIX = H * W
CH2 = C_HID // 2
CPW = CH2 + 16
C_PAD = 256

W4 = float(np.float32(1.0) / (np.float32(4.0) + np.float32(1e-6)))

_GW = 28
_GPW = (B * NPIX) // _GW
_GCH = 32
_GNCH = _GPW // _GCH
_SPT = 49
_RS = 98


def _pos_table():
    p = np.arange(NPIX)
    pr, pc = p // W, p % W
    out = np.empty((NPIX, 4), np.int64)
    k = 0
    for dr in (0, 1):
        for dc in (0, 1):
            out[:, k] = (2 * pr + dr) * (2 * W) + (2 * pc + dc)
            k += 1
    return out


_POS = _pos_table()


def _sc_gather_body(x_hbm, gidx_hbm, out_hbm, idx_v, rows_a, rows_b, acc_v,
                    sem_a, sem_b):
    cid = lax.axis_index("c")
    sid = lax.axis_index("s")
    wid = sid * 2 + cid

    @pl.when(wid < _GW)
    def _():
        pltpu.sync_copy(gidx_hbm.at[wid], idx_v)
        bufs = (rows_a, rows_b)
        sems = (sem_a, sem_b)
        handles = [None] * _GNCH
        handles[0] = pltpu.async_copy(x_hbm.at[idx_v.at[0]], rows_a, sem_a)
        for ck in range(_GNCH):
            handles[ck].wait()
            if ck + 1 < _GNCH:
                handles[ck + 1] = pltpu.async_copy(
                    x_hbm.at[idx_v.at[ck + 1]], bufs[(ck + 1) % 2],
                    sems[(ck + 1) % 2])
            cur = bufs[ck % 2]

            def body(p, carry, cur=cur):
                for cc in range(C_IN // 16):
                    sl = pl.ds(cc * 16, 16)
                    v = (cur[4 * p, sl] + cur[4 * p + 1, sl]) + (
                        cur[4 * p + 2, sl] + cur[4 * p + 3, sl])
                    acc_v[p, sl] = v * W4
                return carry

            lax.fori_loop(0, _GCH, body, 0)
            pltpu.sync_copy(acc_v,
                            out_hbm.at[pl.ds(wid * _GPW + ck * _GCH, _GCH)])


def _sc_scatter_body(mapc_hbm, tok4_hbm, out_hbm, buf_a, buf_b, tidx_v, zero_v,
                     acc_sh, sem_a, sem_b, sem_s):
    cid = lax.axis_index("c")
    sid = lax.axis_index("s")

    def zbody(r, carry):
        for cc in range(CPW // 16):
            zero_v[r, pl.ds(cc * 16, 16)] = jnp.zeros((16,), jnp.float32)
        return carry

    lax.fori_loop(0, _SPT, zbody, 0)
    pltpu.sync_copy(zero_v, acc_sh.at[pl.ds(sid * _RS, _SPT)])
    pltpu.sync_copy(zero_v, acc_sh.at[pl.ds(sid * _RS + _SPT, _SPT)])

    bufs = (buf_a, buf_b)
    sems = (sem_a, sem_b)
    nseg = B
    loads = [None] * nseg
    b0 = cid * (B // 2)
    loads[0] = pltpu.async_copy(
        mapc_hbm.at[b0, 0, pl.ds(sid * _SPT, _SPT)], buf_a, sem_a)
    for seg in range(nseg):
        bb, p = seg // 2, seg % 2
        b = b0 + bb
        if p == 0:
            pltpu.sync_copy(tok4_hbm.at[b, sid], tidx_v)
        loads[seg].wait()
        plsc.subcore_barrier()
        cur = bufs[seg % 2]
        scat = [pltpu.async_copy(cur, acc_sh.at[tidx_v.at[j]], sem_s, add=True)
                for j in range(4)]
        if seg + 1 < nseg:
            nb, np_ = (seg + 1) // 2, (seg + 1) % 2
            loads[seg + 1] = pltpu.async_copy(
                mapc_hbm.at[b0 + nb, np_, pl.ds(sid * _SPT, _SPT)],
                bufs[(seg + 1) % 2], sems[(seg + 1) % 2])
        for h in scat:
            h.wait()
        plsc.subcore_barrier()
        pltpu.sync_copy(acc_sh.at[pl.ds(sid * _RS, _RS)],
                        out_hbm.at[b, p, pl.ds(sid * _RS, _RS)])
        pltpu.sync_copy(zero_v, acc_sh.at[pl.ds(sid * _RS, _SPT)])
        pltpu.sync_copy(zero_v, acc_sh.at[pl.ds(sid * _RS + _SPT, _SPT)])


def _tc_map_body(xmap_ref, fc1t_ref, fc1b_ref, dwt_ref, dwb_ref, out_ref):
    m = jnp.dot(xmap_ref[0].astype(jnp.bfloat16),
                fc1t_ref[...].astype(jnp.bfloat16),
                preferred_element_type=jnp.float32) + fc1b_ref[...]
    m3 = m.reshape(H, W, C_HID)
    zr = jnp.zeros((1, W, C_HID), jnp.float32)
    t = jnp.concatenate([zr, m3, zr], axis=0)
    zc = jnp.zeros((H + 2, 1, C_HID), jnp.float32)
    t = jnp.concatenate([zc, t, zc], axis=1)
    acc = jnp.zeros((H, W, C_HID), jnp.float32)
    for kh in range(3):
        for kw in range(3):
            acc = acc + t[kh:kh + H, kw:kw + W, :] * dwt_ref[3 * kh + kw][None, None, :]
    acc = acc + dwb_ref[...][None]
    res = acc.reshape(NPIX, C_HID)
    pat = jnp.concatenate(
        [jnp.ones((NPIX, 1), jnp.float32), jnp.zeros((NPIX, 15), jnp.float32)],
        axis=1)
    out_ref[0, 0] = jnp.concatenate([res[:, :CH2], pat], axis=1)
    out_ref[0, 1] = jnp.concatenate(
        [res[:, CH2:], jnp.zeros((NPIX, 16), jnp.float32)], axis=1)


def _tc_skip_body(x_ref, fc1t_ref, fc1b_ref, skip_ref, out_ref):
    h = jnp.dot(x_ref[0].astype(jnp.bfloat16), fc1t_ref[...],
                preferred_element_type=jnp.float32) + fc1b_ref[...]
    out_ref[0] = (h * skip_ref[...]).astype(jnp.bfloat16)


def _tc_out_body(hs_ref, tok_ref, fc2t_ref, fc2b_ref, out_ref):
    cnt = lax.slice(tok_ref[0, 0], (0, CH2), (N, CH2 + 1)).astype(jnp.float32)
    val = 1.0 / (cnt + 1e-6)
    tokv = jnp.concatenate(
        [lax.slice(tok_ref[0, 0], (0, 0), (N, CH2)),
         lax.slice(tok_ref[0, 1], (0, 0), (N, CH2))], axis=1).astype(jnp.float32)
    g = hs_ref[0].astype(jnp.float32) + tokv * val
    gel = 0.5 * g * (1.0 + lax.erf(g * np.float32(1.0 / np.sqrt(2.0))))
    out_ref[0] = jnp.dot(gel.astype(jnp.bfloat16), fc2t_ref[...],
                         preferred_element_type=jnp.float32) + fc2b_ref[...]


def kernel(x, idx_token, fc1_w, fc1_b, skip_w, dw_w, dw_b, fc2_w, fc2_b):
    pos = jnp.asarray(_POS.reshape(-1), dtype=jnp.int32)
    a2 = jnp.take(idx_token, pos, axis=1).reshape(B, NPIX, 4).astype(jnp.int32)
    goff = a2 + (jnp.arange(B, dtype=jnp.int32) * N)[:, None, None]
    gidx = goff.reshape(_GW, _GNCH, 4 * _GCH)
    tok4 = a2.reshape(B, 16, _SPT, 4).transpose(0, 1, 3, 2)

    mesh = plsc.VectorSubcoreMesh(core_axis_name="c", subcore_axis_name="s")

    xmap = pl.kernel(
        _sc_gather_body,
        out_type=jax.ShapeDtypeStruct((B * NPIX, C_IN), jnp.float32),
        mesh=mesh,
        scratch_types=[
            pltpu.VMEM((_GNCH, 4 * _GCH), jnp.int32),
            pltpu.VMEM((4 * _GCH, C_PAD), jnp.float32),
            pltpu.VMEM((4 * _GCH, C_PAD), jnp.float32),
            pltpu.VMEM((_GCH, C_IN), jnp.float32),
            pltpu.SemaphoreType.DMA,
            pltpu.SemaphoreType.DMA,
        ],
    )(jnp.pad(x.reshape(B * N, C_IN), ((0, 0), (0, C_PAD - C_IN))), gidx)

    mapc = pl.pallas_call(
        _tc_map_body,
        grid=(B,),
        in_specs=[
            pl.BlockSpec((1, NPIX, C_IN), lambda b: (b, 0, 0)),
            pl.BlockSpec((C_IN, C_HID), lambda b: (0, 0)),
            pl.BlockSpec((1, C_HID), lambda b: (0, 0)),
            pl.BlockSpec((9, C_HID), lambda b: (0, 0)),
            pl.BlockSpec((1, C_HID), lambda b: (0, 0)),
        ],
        out_specs=pl.BlockSpec((1, 2, NPIX, CPW), lambda b: (b, 0, 0, 0)),
        out_shape=jax.ShapeDtypeStruct((B, 2, NPIX, CPW), jnp.float32),
    )(
        xmap.reshape(B, NPIX, C_IN),
        fc1_w.T,
        fc1_b.reshape(1, C_HID),
        dw_w.reshape(C_HID, 9).T,
        dw_b.reshape(1, C_HID),
    )

    tok_acc = pl.kernel(
        _sc_scatter_body,
        out_type=jax.ShapeDtypeStruct((B, 2, N, CPW), jnp.float32),
        mesh=mesh,
        compiler_params=pltpu.CompilerParams(use_tc_tiling_on_sc=False),
        scratch_types=[
            pltpu.VMEM((_SPT, CPW), jnp.float32),
            pltpu.VMEM((_SPT, CPW), jnp.float32),
            pltpu.VMEM((4, _SPT), jnp.int32),
            pltpu.VMEM((_SPT, CPW), jnp.float32),
            pltpu.VMEM_SHARED((N, CPW), jnp.float32),
            pltpu.SemaphoreType.DMA,
            pltpu.SemaphoreType.DMA,
            pltpu.SemaphoreType.DMA,
        ],
    )(mapc, tok4)

    h_skip = pl.pallas_call(
        _tc_skip_body,
        grid=(B,),
        in_specs=[
            pl.BlockSpec((1, N, C_IN), lambda b: (b, 0, 0)),
            pl.BlockSpec((C_IN, C_HID), lambda b: (0, 0)),
            pl.BlockSpec((1, C_HID), lambda b: (0, 0)),
            pl.BlockSpec((1, C_HID), lambda b: (0, 0)),
        ],
        out_specs=pl.BlockSpec((1, N, C_HID), lambda b: (b, 0, 0)),
        out_shape=jax.ShapeDtypeStruct((B, N, C_HID), jnp.bfloat16),
    )(
        x,
        fc1_w.T.astype(jnp.bfloat16),
        fc1_b.reshape(1, C_HID),
        skip_w.reshape(1, C_HID),
    )

    out = pl.pallas_call(
        _tc_out_body,
        grid=(B,),
        in_specs=[
            pl.BlockSpec((1, N, C_HID), lambda b: (b, 0, 0)),
            pl.BlockSpec((1, 2, N, CPW), lambda b: (b, 0, 0, 0)),
            pl.BlockSpec((C_HID, C_OUT), lambda b: (0, 0)),
            pl.BlockSpec((1, C_OUT), lambda b: (0, 0)),
        ],
        out_specs=pl.BlockSpec((1, N, C_OUT), lambda b: (b, 0, 0)),
        out_shape=jax.ShapeDtypeStruct((B, N, C_OUT), jnp.float32),
    )(
        h_skip,
        tok_acc,
        fc2_w.T.astype(jnp.bfloat16),
        fc2_b.reshape(1, C_OUT),
    )
    return out
